# baseline (speedup 1.0000x reference)
import functools

import jax
import jax.numpy as jnp
from jax import lax
from jax.experimental import pallas as pl
from jax.experimental.pallas import tpu as pltpu
from jax.experimental.pallas import tpu_sc as plsc

B = 4
S = 8192
D = 768
L = 16
NC = 2
NS = 16
NW = NC * NS
ROWS_PER_W = S // NW
CHUNK = 16
N_CHUNKS = ROWS_PER_W // CHUNK
CGROUPS = D // L


def _pe_add(x_hbm, enc_hbm, out_hbm, *scr):
    enc_v = [scr[0], scr[1]]
    xv = [[scr[2 + 2 * b], scr[2 + 2 * b + 1]] for b in range(B)]
    esem = [scr[10], scr[11]]
    isem = [[scr[12 + 2 * b], scr[12 + 2 * b + 1]] for b in range(B)]
    osem = [[scr[20 + 2 * b], scr[20 + 2 * b + 1]] for b in range(B)]
    wid = lax.axis_index("s") * NC + lax.axis_index("c")
    base = wid * ROWS_PER_W

    def start_enc(ci, p):
        pltpu.async_copy(
            enc_hbm.at[pl.ds(base + ci * CHUNK, CHUNK)], enc_v[p], esem[p])

    def wait_enc(p):
        pltpu.make_async_copy(
            enc_hbm.at[pl.ds(0, CHUNK)], enc_v[p], esem[p]).wait()

    def start_in(ci, b, p):
        pltpu.async_copy(
            x_hbm.at[b, pl.ds(base + ci * CHUNK, CHUNK)], xv[b][p],
            isem[b][p])

    def wait_in(b, p):
        pltpu.make_async_copy(
            x_hbm.at[b, pl.ds(0, CHUNK)], xv[b][p], isem[b][p]).wait()

    def wait_out(b, p):
        pltpu.make_async_copy(
            xv[b][p], out_hbm.at[b, pl.ds(0, CHUNK)], osem[b][p]).wait()

    def add_chunk(buf, ev):
        @plsc.parallel_loop(0, CHUNK, 1, unroll=4)
        def _(r):
            for c in range(CGROUPS):
                sl = pl.ds(c * L, L)
                plsc.addupdate(buf.at[r, sl], ev[r, sl])

    def phase(ci, p):
        s0 = base + ci * CHUNK
        ci_next = jnp.minimum(ci + 1, N_CHUNKS - 1)
        start_enc(ci_next, 1 - p)
        for b in range(B):
            @pl.when(ci > 0)
            def _():
                wait_out(b, 1 - p)

            start_in(ci_next, b, 1 - p)
        wait_enc(p)
        for b in range(B):
            wait_in(b, p)
            add_chunk(xv[b][p], enc_v[p])
            pltpu.async_copy(
                xv[b][p], out_hbm.at[b, pl.ds(s0, CHUNK)], osem[b][p])

    start_enc(0, 0)
    for b in range(B):
        start_in(0, b, 0)

    def pair_body(ci2, carry):
        phase(2 * ci2, 0)
        phase(2 * ci2 + 1, 1)
        return carry

    lax.fori_loop(0, N_CHUNKS // 2, pair_body, 0)
    wait_enc(0)
    for b in range(B):
        wait_out(b, 1)
        wait_in(b, 0)


@jax.jit
def kernel(x, encoding):
    mesh = plsc.VectorSubcoreMesh(core_axis_name="c", subcore_axis_name="s")
    scratch = [pltpu.VMEM((CHUNK, D), jnp.float32)] * 2
    scratch += [pltpu.VMEM((CHUNK, D), jnp.float32)] * 8
    scratch += [pltpu.SemaphoreType.DMA] * 18
    f = functools.partial(
        pl.kernel,
        mesh=mesh,
        out_type=jax.ShapeDtypeStruct((B, S, D), jnp.float32),
        scratch_types=scratch,
    )(_pe_add)
    return f(x, encoding)


# speedup vs baseline: 1.4701x; 1.4701x over previous
import functools

import jax
import jax.numpy as jnp
from jax import lax
from jax.experimental import pallas as pl
from jax.experimental.pallas import tpu as pltpu
from jax.experimental.pallas import tpu_sc as plsc

B = 4
S = 8192
D = 768
L = 16
NC = 2
NS = 16
NW = NC * NS
ROWS_PER_W = S // NW
CHUNK = 16
N_CHUNKS = ROWS_PER_W // CHUNK
CGROUPS = D // L


def _pe_add(x_hbm, enc_hbm, out_hbm, *scr):
    enc_v = [scr[0], scr[1]]
    xv = [[scr[2 + 2 * b], scr[2 + 2 * b + 1]] for b in range(B)]
    esem = [scr[10], scr[11]]
    isem = [[scr[12 + 2 * b], scr[12 + 2 * b + 1]] for b in range(B)]
    osem = [[scr[20 + 2 * b], scr[20 + 2 * b + 1]] for b in range(B)]
    wid = lax.axis_index("s") * NC + lax.axis_index("c")
    base = wid * ROWS_PER_W

    def start_enc(ci, p):
        pltpu.async_copy(
            enc_hbm.at[pl.ds(base + ci * CHUNK, CHUNK)], enc_v[p], esem[p])

    def wait_enc(p):
        pltpu.make_async_copy(
            enc_hbm.at[pl.ds(0, CHUNK)], enc_v[p], esem[p]).wait()

    def start_in(ci, b, p):
        pltpu.async_copy(
            x_hbm.at[b, pl.ds(base + ci * CHUNK, CHUNK)], xv[b][p],
            isem[b][p])

    def wait_in(b, p):
        pltpu.make_async_copy(
            x_hbm.at[b, pl.ds(0, CHUNK)], xv[b][p], isem[b][p]).wait()

    def wait_out(b, p):
        pltpu.make_async_copy(
            xv[b][p], out_hbm.at[b, pl.ds(0, CHUNK)], osem[b][p]).wait()

    def add_chunk(buf, ev):
        @plsc.parallel_loop(0, CHUNK, 1, unroll=1)
        def _(r):
            for c in range(CGROUPS):
                sl = pl.ds(c * L, L)
                plsc.addupdate(buf.at[r, sl], ev[r, sl])

    def phase(ci, p):
        s0 = base + ci * CHUNK
        ci_next = jnp.minimum(ci + 1, N_CHUNKS - 1)
        start_enc(ci_next, 1 - p)
        for b in range(B):
            @pl.when(ci > 0)
            def _():
                wait_out(b, 1 - p)

            start_in(ci_next, b, 1 - p)
        wait_enc(p)
        for b in range(B):
            wait_in(b, p)
            add_chunk(xv[b][p], enc_v[p])
            pltpu.async_copy(
                xv[b][p], out_hbm.at[b, pl.ds(s0, CHUNK)], osem[b][p])

    start_enc(0, 0)
    for b in range(B):
        start_in(0, b, 0)

    def pair_body(ci2, carry):
        phase(2 * ci2, 0)
        phase(2 * ci2 + 1, 1)
        return carry

    lax.fori_loop(0, N_CHUNKS // 2, pair_body, 0)
    wait_enc(0)
    for b in range(B):
        wait_out(b, 1)
        wait_in(b, 0)


@jax.jit
def kernel(x, encoding):
    mesh = plsc.VectorSubcoreMesh(core_axis_name="c", subcore_axis_name="s")
    scratch = [pltpu.VMEM((CHUNK, D), jnp.float32)] * 2
    scratch += [pltpu.VMEM((CHUNK, D), jnp.float32)] * 8
    scratch += [pltpu.SemaphoreType.DMA] * 18
    f = functools.partial(
        pl.kernel,
        mesh=mesh,
        out_type=jax.ShapeDtypeStruct((B, S, D), jnp.float32),
        scratch_types=scratch,
    )(_pe_add)
    return f(x, encoding)


# speedup vs baseline: 1.5060x; 1.0244x over previous
import functools

import jax
import jax.numpy as jnp
from jax import lax
from jax.experimental import pallas as pl
from jax.experimental.pallas import tpu as pltpu
from jax.experimental.pallas import tpu_sc as plsc

B = 4
S = 8192
D = 768
L = 16
NC = 2
NS = 16
NW = NC * NS
ROWS_PER_W = S // NW
CHUNK = 16
N_CHUNKS = ROWS_PER_W // CHUNK
CGROUPS = D // L


def _pe_add(x_hbm, enc_hbm, out_hbm, *scr):
    enc_v = [scr[0], scr[1]]
    xv = [[scr[2 + 2 * b], scr[2 + 2 * b + 1]] for b in range(B)]
    esem = [scr[10], scr[11]]
    isem = [[scr[12 + 2 * b], scr[12 + 2 * b + 1]] for b in range(B)]
    osem = [[scr[20 + 2 * b], scr[20 + 2 * b + 1]] for b in range(B)]
    wid = lax.axis_index("s") * NC + lax.axis_index("c")
    base = wid * ROWS_PER_W

    def start_enc(ci, p):
        pltpu.async_copy(
            enc_hbm.at[pl.ds(base + ci * CHUNK, CHUNK)], enc_v[p], esem[p])

    def wait_enc(p):
        pltpu.make_async_copy(
            enc_hbm.at[pl.ds(0, CHUNK)], enc_v[p], esem[p]).wait()

    def start_in(ci, b, p):
        pltpu.async_copy(
            x_hbm.at[b, pl.ds(base + ci * CHUNK, CHUNK)], xv[b][p],
            isem[b][p])

    def wait_in(b, p):
        pltpu.make_async_copy(
            x_hbm.at[b, pl.ds(0, CHUNK)], xv[b][p], isem[b][p]).wait()

    def wait_out(b, p):
        pltpu.make_async_copy(
            xv[b][p], out_hbm.at[b, pl.ds(0, CHUNK)], osem[b][p]).wait()

    def add_chunk(buf, ev):
        @plsc.parallel_loop(0, 2 * CHUNK, 1, unroll=1)
        def _(i):
            r = i // 2
            h = (i % 2) * (D // 2)
            for c in range(CGROUPS // 2):
                sl = pl.ds(h + c * L, L)
                plsc.addupdate(buf.at[r, sl], ev[r, sl])

    def phase(ci, p):
        s0 = base + ci * CHUNK
        ci_next = jnp.minimum(ci + 1, N_CHUNKS - 1)
        start_enc(ci_next, 1 - p)
        for b in range(B):
            @pl.when(ci > 0)
            def _():
                wait_out(b, 1 - p)

            start_in(ci_next, b, 1 - p)
        wait_enc(p)
        for b in range(B):
            wait_in(b, p)
            add_chunk(xv[b][p], enc_v[p])
            pltpu.async_copy(
                xv[b][p], out_hbm.at[b, pl.ds(s0, CHUNK)], osem[b][p])

    start_enc(0, 0)
    for b in range(B):
        start_in(0, b, 0)

    def pair_body(ci2, carry):
        phase(2 * ci2, 0)
        phase(2 * ci2 + 1, 1)
        return carry

    lax.fori_loop(0, N_CHUNKS // 2, pair_body, 0)
    wait_enc(0)
    for b in range(B):
        wait_out(b, 1)
        wait_in(b, 0)


@jax.jit
def kernel(x, encoding):
    mesh = plsc.VectorSubcoreMesh(core_axis_name="c", subcore_axis_name="s")
    scratch = [pltpu.VMEM((CHUNK, D), jnp.float32)] * 2
    scratch += [pltpu.VMEM((CHUNK, D), jnp.float32)] * 8
    scratch += [pltpu.SemaphoreType.DMA] * 18
    f = functools.partial(
        pl.kernel,
        mesh=mesh,
        out_type=jax.ShapeDtypeStruct((B, S, D), jnp.float32),
        scratch_types=scratch,
    )(_pe_add)
    return f(x, encoding)


# speedup vs baseline: 1.5214x; 1.0102x over previous
import functools

import jax
import jax.numpy as jnp
from jax import lax
from jax.experimental import pallas as pl
from jax.experimental.pallas import tpu as pltpu
from jax.experimental.pallas import tpu_sc as plsc

B = 4
S = 8192
D = 768
L = 16
NC = 2
NS = 16
NW = NC * NS
ROWS_PER_W = S // NW
CHUNK = 16
N_CHUNKS = ROWS_PER_W // CHUNK
CGROUPS = D // L


def _pe_add(x_hbm, enc_hbm, out_hbm, *scr):
    enc_v = [scr[0], scr[1]]
    xv = [[scr[2 + 2 * b], scr[2 + 2 * b + 1]] for b in range(B)]
    esem = [scr[10], scr[11]]
    isem = [[scr[12 + 2 * b], scr[12 + 2 * b + 1]] for b in range(B)]
    osem = [[scr[20 + 2 * b], scr[20 + 2 * b + 1]] for b in range(B)]
    wid = lax.axis_index("s") * NC + lax.axis_index("c")
    base = wid * ROWS_PER_W

    def start_enc(ci, p):
        pltpu.async_copy(
            enc_hbm.at[pl.ds(base + ci * CHUNK, CHUNK)], enc_v[p], esem[p])

    def wait_enc(p):
        pltpu.make_async_copy(
            enc_hbm.at[pl.ds(0, CHUNK)], enc_v[p], esem[p]).wait()

    def start_in(ci, b, p):
        pltpu.async_copy(
            x_hbm.at[b, pl.ds(base + ci * CHUNK, CHUNK)], xv[b][p],
            isem[b][p])

    def wait_in(b, p):
        pltpu.make_async_copy(
            x_hbm.at[b, pl.ds(0, CHUNK)], xv[b][p], isem[b][p]).wait()

    def wait_out(b, p):
        pltpu.make_async_copy(
            xv[b][p], out_hbm.at[b, pl.ds(0, CHUNK)], osem[b][p]).wait()

    def add_chunk(buf, ev):
        @plsc.parallel_loop(0, 4 * CHUNK, 1, unroll=1)
        def _(i):
            r = i // 4
            h = (i % 4) * (D // 4)
            for c in range(CGROUPS // 4):
                sl = pl.ds(h + c * L, L)
                plsc.addupdate(buf.at[r, sl], ev[r, sl])

    def phase(ci, p):
        s0 = base + ci * CHUNK
        ci_next = jnp.minimum(ci + 1, N_CHUNKS - 1)
        start_enc(ci_next, 1 - p)
        for b in range(B):
            @pl.when(ci > 0)
            def _():
                wait_out(b, 1 - p)

            start_in(ci_next, b, 1 - p)
        wait_enc(p)
        for b in range(B):
            wait_in(b, p)
            add_chunk(xv[b][p], enc_v[p])
            pltpu.async_copy(
                xv[b][p], out_hbm.at[b, pl.ds(s0, CHUNK)], osem[b][p])

    start_enc(0, 0)
    for b in range(B):
        start_in(0, b, 0)

    def pair_body(ci2, carry):
        phase(2 * ci2, 0)
        phase(2 * ci2 + 1, 1)
        return carry

    lax.fori_loop(0, N_CHUNKS // 2, pair_body, 0)
    wait_enc(0)
    for b in range(B):
        wait_out(b, 1)
        wait_in(b, 0)


@jax.jit
def kernel(x, encoding):
    mesh = plsc.VectorSubcoreMesh(core_axis_name="c", subcore_axis_name="s")
    scratch = [pltpu.VMEM((CHUNK, D), jnp.float32)] * 2
    scratch += [pltpu.VMEM((CHUNK, D), jnp.float32)] * 8
    scratch += [pltpu.SemaphoreType.DMA] * 18
    f = functools.partial(
        pl.kernel,
        mesh=mesh,
        out_type=jax.ShapeDtypeStruct((B, S, D), jnp.float32),
        scratch_types=scratch,
    )(_pe_add)
    return f(x, encoding)
